# Initial kernel scaffold; baseline (speedup 1.0000x reference)
import jax
import jax.numpy as jnp
from jax.experimental import pallas as pl


def kernel(type_indices, type_embedding):
    raise NotImplementedError("write your pallas kernel here")


# speedup vs baseline: 1.2700x; 1.2700x over previous
import functools

import jax
import jax.numpy as jnp
from jax import lax
from jax.experimental import pallas as pl
from jax.experimental.pallas import tpu as pltpu
from jax.experimental.pallas import tpu_sc as plsc

NUM_NODES = 100000
EMBED_DIM = 64
CHUNK = 128
NFULL = NUM_NODES // CHUNK
TAIL = NUM_NODES - NFULL * CHUNK
NUM_WORKERS = 32

_mesh = plsc.VectorSubcoreMesh(core_axis_name="c", subcore_axis_name="s")


@functools.partial(
    pl.kernel,
    mesh=_mesh,
    compiler_params=pltpu.CompilerParams(use_tc_tiling_on_sc=False),
    out_type=jax.ShapeDtypeStruct((NUM_NODES, EMBED_DIM), jnp.float32),
    scratch_types=[
        pltpu.VMEM((CHUNK,), jnp.int32),
        pltpu.VMEM((CHUNK, EMBED_DIM), jnp.float32),
        pltpu.VMEM((TAIL,), jnp.int32),
        pltpu.VMEM((TAIL, EMBED_DIM), jnp.float32),
        pltpu.SemaphoreType.DMA,
    ],
)
def _gather_kernel(idx_hbm, table_hbm, out_hbm, idx_v, rows_v, idx_t,
                   rows_t, sem):
    wid = lax.axis_index("s") * 2 + lax.axis_index("c")
    niter = NFULL // NUM_WORKERS + 1

    def body(i, carry):
        chunk = i * NUM_WORKERS + wid

        @pl.when(chunk < NFULL)
        def _():
            base = chunk * CHUNK
            pltpu.sync_copy(idx_hbm.at[pl.ds(base, CHUNK)], idx_v)
            pltpu.async_copy(table_hbm.at[idx_v], rows_v, sem).wait()
            pltpu.sync_copy(rows_v, out_hbm.at[pl.ds(base, CHUNK)])

        @pl.when(chunk == NFULL)
        def _():
            base = NFULL * CHUNK
            pltpu.sync_copy(idx_hbm.at[pl.ds(base, TAIL)], idx_t)
            pltpu.async_copy(table_hbm.at[idx_t], rows_t, sem).wait()
            pltpu.sync_copy(rows_t, out_hbm.at[pl.ds(base, TAIL)])

        return carry

    lax.fori_loop(0, niter, body, 0)


def kernel(type_indices, type_embedding):
    return _gather_kernel(type_indices.astype(jnp.int32), type_embedding)
